# Initial kernel scaffold; baseline (speedup 1.0000x reference)
import jax
import jax.numpy as jnp
from jax.experimental import pallas as pl


def kernel(x, edge_index, W_self0, W_nbr0, b0, g0, be0, W_self1, W_nbr1, b1, g1, be1, W_self2, W_nbr2, b2, g2, be2, W_out, b_out):
    raise NotImplementedError("write your pallas kernel here")


# speedup vs baseline: 2.3925x; 2.3925x over previous
import functools

import jax
import jax.numpy as jnp
from jax import lax
from jax.experimental import pallas as pl
from jax.experimental.pallas import tpu as pltpu
from jax.experimental.pallas import tpu_sc as plsc

N_PAD = 10240
CB = 8
BATCH = 128
TILES = 16
ROWS_PER_TILE = N_PAD // TILES
DST_PAD = 10200


def _make_segsum(e_pad: int):
    e_tile = e_pad // TILES
    nb = e_tile // BATCH
    mesh = plsc.VectorSubcoreMesh(core_axis_name="c", subcore_axis_name="s")

    @functools.partial(
        pl.kernel,
        mesh=mesh,
        out_type=jax.ShapeDtypeStruct((N_PAD, CB, 128), jnp.float32),
        scratch_types=[
            pltpu.VMEM((BATCH,), jnp.int32),
            pltpu.VMEM((BATCH,), jnp.int32),
            pltpu.VMEM((BATCH, 128), jnp.float32),
            pltpu.VMEM((32, 128), jnp.float32),
            pltpu.VMEM_SHARED((N_PAD, 128), jnp.float32),
            pltpu.SemaphoreType.DMA,
        ],
    )
    def segsum(hw_hbm, gidx_hbm, dst_hbm, out_hbm,
               gbuf, dbuf, rows, zbuf, acc, sem):
        c = lax.axis_index("c")
        s = lax.axis_index("s")

        def zfill(r, carry):
            for j in range(8):
                zbuf[r, pl.ds(j * 16, 16)] = jnp.zeros((16,), jnp.float32)
            return carry
        lax.fori_loop(0, 32, zfill, 0)

        for k in range(CB // 2):
            cb = c * (CB // 2) + k

            for j in range(ROWS_PER_TILE // 32):
                pltpu.sync_copy(zbuf, acc.at[pl.ds(s * ROWS_PER_TILE + j * 32, 32)])
            plsc.subcore_barrier()

            def body(b, carry):
                base = s * e_tile + b * BATCH
                pltpu.sync_copy(gidx_hbm.at[cb, pl.ds(base, BATCH)], gbuf)
                pltpu.sync_copy(dst_hbm.at[pl.ds(base, BATCH)], dbuf)
                pltpu.async_copy(hw_hbm.at[gbuf], rows, sem).wait()
                pltpu.sync_copy(rows, acc.at[dbuf], add=True)
                return carry
            lax.fori_loop(0, nb, body, 0)
            plsc.subcore_barrier()

            pltpu.sync_copy(acc.at[pl.ds(s * ROWS_PER_TILE, ROWS_PER_TILE)],
                            out_hbm.at[pl.ds(s * ROWS_PER_TILE, ROWS_PER_TILE), cb])

    return segsum


def _mm_body(h_ref, w_ref, o_ref):
    o_ref[...] = jnp.dot(h_ref[...], w_ref[...],
                         preferred_element_type=jnp.float32)


def _mm(h, w, bm=512):
    m, k = h.shape
    n = w.shape[1]
    return pl.pallas_call(
        _mm_body,
        grid=(m // bm,),
        in_specs=[pl.BlockSpec((bm, k), lambda i: (i, 0)),
                  pl.BlockSpec((k, n), lambda i: (0, 0))],
        out_specs=pl.BlockSpec((bm, n), lambda i: (i, 0)),
        out_shape=jax.ShapeDtypeStruct((m, n), jnp.float32),
    )(h, w)


def _mm_bias_body(h_ref, w_ref, b_ref, o_ref):
    o_ref[...] = (jnp.dot(h_ref[...], w_ref[...],
                          preferred_element_type=jnp.float32)
                  + b_ref[...])


def _mm_bias(h, w, b, bm=512):
    m, k = h.shape
    n = w.shape[1]
    return pl.pallas_call(
        _mm_bias_body,
        grid=(m // bm,),
        in_specs=[pl.BlockSpec((bm, k), lambda i: (i, 0)),
                  pl.BlockSpec((k, n), lambda i: (0, 0)),
                  pl.BlockSpec((1, n), lambda i: (0, 0))],
        out_specs=pl.BlockSpec((bm, n), lambda i: (i, 0)),
        out_shape=jax.ShapeDtypeStruct((m, n), jnp.float32),
    )(h, w, b)


def _combine_body(residual, h_ref, w_ref, agg_ref, b_ref, g_ref, be_ref, o_ref):
    z = (jnp.dot(h_ref[...], w_ref[...], preferred_element_type=jnp.float32)
         + agg_ref[...] + b_ref[...])
    if residual:
        z = z + h_ref[...]
    mu = jnp.mean(z, axis=1, keepdims=True)
    d = z - mu
    var = jnp.mean(d * d, axis=1, keepdims=True)
    y = d * lax.rsqrt(var + 1e-5) * g_ref[...] + be_ref[...]
    o_ref[...] = jnp.where(y > 0, y, 0.01 * y)


def _combine(h, w, agg, b, g, be, residual, bm=512):
    m, k = h.shape
    n = w.shape[1]
    return pl.pallas_call(
        functools.partial(_combine_body, residual),
        grid=(m // bm,),
        in_specs=[pl.BlockSpec((bm, k), lambda i: (i, 0)),
                  pl.BlockSpec((k, n), lambda i: (0, 0)),
                  pl.BlockSpec((bm, n), lambda i: (i, 0)),
                  pl.BlockSpec((1, n), lambda i: (0, 0)),
                  pl.BlockSpec((1, n), lambda i: (0, 0)),
                  pl.BlockSpec((1, n), lambda i: (0, 0))],
        out_specs=pl.BlockSpec((bm, n), lambda i: (i, 0)),
        out_shape=jax.ShapeDtypeStruct((m, n), jnp.float32),
    )(h, w, agg, b, g, be)


def kernel(x, edge_index, W_self0, W_nbr0, b0, g0, be0,
           W_self1, W_nbr1, b1, g1, be1,
           W_self2, W_nbr2, b2, g2, be2, W_out, b_out):
    n, in_dim = x.shape
    e = edge_index.shape[1]
    h_dim = W_self0.shape[1]
    out_dim = W_out.shape[1]

    e_pad = ((e + TILES * BATCH - 1) // (TILES * BATCH)) * (TILES * BATCH)
    src = jnp.pad(edge_index[0], (0, e_pad - e))
    dst = jnp.pad(edge_index[1], (0, e_pad - e), constant_values=DST_PAD)
    gidx = src[None, :] * CB + jnp.arange(CB, dtype=jnp.int32)[:, None]

    segsum = _make_segsum(e_pad)

    hp = jnp.pad(x, ((0, N_PAD - n), (0, 0)))
    params = [(W_self0, W_nbr0, b0, g0, be0),
              (W_self1, W_nbr1, b1, g1, be1),
              (W_self2, W_nbr2, b2, g2, be2)]
    for i, (ws, wn, b, g, be) in enumerate(params):
        hw = _mm(hp, wn)
        agg = segsum(hw.reshape(N_PAD * CB, 128), gidx, dst)
        agg = agg.reshape(N_PAD, h_dim)
        hp = _combine(hp, ws, agg, b.reshape(1, h_dim), g.reshape(1, h_dim),
                      be.reshape(1, h_dim), residual=(i > 0))

    wo = jnp.pad(W_out, ((0, 0), (0, 128 - out_dim)))
    bo = jnp.pad(b_out, (0, 128 - out_dim)).reshape(1, 128)
    out = _mm_bias(hp, wo, bo)
    return out[:n, :out_dim]
